# Initial kernel scaffold; baseline (speedup 1.0000x reference)
import jax
import jax.numpy as jnp
from jax.experimental import pallas as pl


def kernel(x, conv1_w, conv1_b, score_w, score_b, loc_w, loc_b, img_size):
    raise NotImplementedError("write your pallas kernel here")


# speedup vs baseline: 16.0267x; 16.0267x over previous
import numpy as np
import jax
import jax.numpy as jnp
from jax.experimental import pallas as pl
from jax.experimental.pallas import tpu as pltpu

_FEAT_STRIDE = 16
_N_POST = 2000
_N_PRE = 12000
_NMS_THRESH = 0.7
_MIN_SIZE = 16.0
_LANES = 128


def _anchor_base_np(base_size=16, ratios=(0.5, 1.0, 2.0), anchor_scales=(8, 16, 32)):
    py = base_size / 2.0
    px = base_size / 2.0
    ab = np.zeros((len(ratios) * len(anchor_scales), 4), dtype=np.float32)
    for i, r in enumerate(ratios):
        for j, s in enumerate(anchor_scales):
            h = base_size * s * np.sqrt(r)
            w = base_size * s * np.sqrt(1.0 / r)
            k = i * len(anchor_scales) + j
            ab[k, 0] = py - h / 2.0
            ab[k, 1] = px - w / 2.0
            ab[k, 2] = py + h / 2.0
            ab[k, 3] = px + w / 2.0
    return ab


def _anchors_np(height, width):
    ab = _anchor_base_np()
    shift_y = np.arange(0, height * _FEAT_STRIDE, _FEAT_STRIDE)
    shift_x = np.arange(0, width * _FEAT_STRIDE, _FEAT_STRIDE)
    sx, sy = np.meshgrid(shift_x, shift_y, indexing='ij')
    shift = np.stack((sy.ravel(), sx.ravel(), sy.ravel(), sx.ravel()), axis=1)
    A = ab.shape[0]
    K = shift.shape[0]
    anchor = ab.reshape((1, A, 4)) + shift.reshape((K, 1, 4))
    return anchor.reshape((K * A, 4)).astype(np.float32)


def _conv2d(x, w, b, pad):
    y = jax.lax.conv_general_dilated(x, w, (1, 1), pad,
                                     dimension_numbers=('NCHW', 'OIHW', 'NCHW'))
    return y + b[None, :, None, None]


def _proposal_body(img_ref, dy_ref, dx_ref, dh_ref, dw_ref, fg_ref,
                   ay1_ref, ax1_ref, ay2_ref, ax2_ref,
                   rois_ref,
                   y1_s, x1_s, y2_s, x2_s, ar_s, s_s, idx_s,
                   *, n_total, n_pre, n_post):
    H = img_ref[0, 0]
    W = img_ref[0, 1]
    rows = y1_s.shape[0]
    neg_inf = jnp.float32(-jnp.inf)

    ay1 = ay1_ref[...]
    ax1 = ax1_ref[...]
    src_h = ay2_ref[...] - ay1
    src_w = ax2_ref[...] - ax1
    src_cy = ay1 + 0.5 * src_h
    src_cx = ax1 + 0.5 * src_w
    cy = dy_ref[...] * src_h + src_cy
    cx = dx_ref[...] * src_w + src_cx
    h = jnp.exp(dh_ref[...]) * src_h
    w = jnp.exp(dw_ref[...]) * src_w
    y1 = jnp.clip(cy - 0.5 * h, 0.0, H)
    x1 = jnp.clip(cx - 0.5 * w, 0.0, W)
    y2 = jnp.clip(cy + 0.5 * h, 0.0, H)
    x2 = jnp.clip(cx + 0.5 * w, 0.0, W)
    hs = y2 - y1
    ws = x2 - x1
    valid = (hs >= _MIN_SIZE) & (ws >= _MIN_SIZE)
    s = jnp.where(valid, fg_ref[...], neg_inf)

    idx = (jax.lax.broadcasted_iota(jnp.int32, (rows, _LANES), 0) * _LANES
           + jax.lax.broadcasted_iota(jnp.int32, (rows, _LANES), 1))

    sbits = jax.lax.bitcast_convert_type(s, jnp.int32)
    key = jnp.where(s >= 0.0, sbits + 1, 0)

    k_target = jnp.int32(n_pre)

    def _bs_val(_, lohi):
        lo, hi = lohi
        mid = (lo + hi) // 2
        cnt = jnp.sum((key >= mid).astype(jnp.int32))
        big = cnt >= k_target
        return jnp.where(big, mid, lo), jnp.where(big, hi, mid)

    lo0 = jnp.int32(0)
    hi0 = jnp.int32((1 << 30) + 2)
    lo, hi = jax.lax.fori_loop(0, 31, _bs_val, (lo0, hi0))
    thr = lo
    cnt_gt = jnp.sum((key > thr).astype(jnp.int32))
    need_eq = k_target - cnt_gt
    ties = key == thr

    def _bs_idx(_, lohi):
        lo2, hi2 = lohi
        mid = (lo2 + hi2) // 2
        cnt = jnp.sum((ties & (idx < mid)).astype(jnp.int32))
        enough = cnt >= need_eq
        return jnp.where(enough, lo2, mid), jnp.where(enough, mid, hi2)

    lo2, hi2 = jax.lax.fori_loop(0, 16, _bs_idx, (jnp.int32(0), jnp.int32(n_total)))
    cut = hi2
    eligible = (key > thr) | (ties & (idx < cut))
    s = jnp.where(eligible, s, neg_inf)

    y1_s[...] = y1
    x1_s[...] = x1
    y2_s[...] = y2
    x2_s[...] = x2
    ar_s[...] = hs * ws
    s_s[...] = s
    idx_s[...] = idx

    lane_iota = jax.lax.broadcasted_iota(jnp.int32, (1, _LANES), 1)

    def _pick_coord(ref, r, lane):
        row = ref[pl.ds(r, 1), :]
        return jnp.max(jnp.where(lane_iota == lane, row, neg_inf))

    def _nms_iter(i, carry):
        fy1, fx1, fy2, fx2 = carry
        s_cur = s_s[...]
        m = jnp.max(s_cur)
        pickmask = s_cur == m
        pick = jnp.min(jnp.where(pickmask, idx_s[...], jnp.int32(1 << 30)))
        r = pick // _LANES
        lane = pick - r * _LANES
        by1 = _pick_coord(y1_s, r, lane)
        bx1 = _pick_coord(x1_s, r, lane)
        by2 = _pick_coord(y2_s, r, lane)
        bx2 = _pick_coord(x2_s, r, lane)
        barea = _pick_coord(ar_s, r, lane)

        first = i == 0
        fy1 = jnp.where(first, by1, fy1)
        fx1 = jnp.where(first, bx1, fx1)
        fy2 = jnp.where(first, by2, fy2)
        fx2 = jnp.where(first, bx2, fx2)
        exhausted = m == neg_inf
        oy1 = jnp.where(exhausted, fy1, by1)
        ox1 = jnp.where(exhausted, fx1, bx1)
        oy2 = jnp.where(exhausted, fy2, by2)
        ox2 = jnp.where(exhausted, fx2, bx2)

        lane_f = lane_iota
        row_out = (oy1 * (lane_f == 0).astype(jnp.float32)
                   + ox1 * (lane_f == 1).astype(jnp.float32)
                   + oy2 * (lane_f == 2).astype(jnp.float32)
                   + ox2 * (lane_f == 3).astype(jnp.float32))
        rois_ref[pl.ds(i, 1), :] = row_out

        yy1 = jnp.maximum(y1_s[...], by1)
        xx1 = jnp.maximum(x1_s[...], bx1)
        yy2 = jnp.minimum(y2_s[...], by2)
        xx2 = jnp.minimum(x2_s[...], bx2)
        inter = jnp.maximum(yy2 - yy1, 0.0) * jnp.maximum(xx2 - xx1, 0.0)
        iou = inter / (ar_s[...] + barea - inter + jnp.float32(1e-9))
        kill = (iou > _NMS_THRESH) | (idx_s[...] == pick)
        s_s[...] = jnp.where(kill, neg_inf, s_cur)
        return fy1, fx1, fy2, fx2

    z = jnp.float32(0.0)
    jax.lax.fori_loop(0, n_post, _nms_iter, (z, z, z, z))


def _proposal_pallas(locs, fg, anchor, img_size):
    n_total = locs.shape[0]
    assert n_total % _LANES == 0
    rows = n_total // _LANES
    n_pre = min(_N_PRE, n_total)

    lt = locs.T.reshape(4, rows, _LANES)
    at = anchor.T.reshape(4, rows, _LANES)
    fg2 = fg.reshape(rows, _LANES)
    img_row = jnp.zeros((1, _LANES), jnp.float32).at[0, :2].set(
        img_size.astype(jnp.float32))

    import functools
    body = functools.partial(_proposal_body, n_total=n_total, n_pre=n_pre,
                             n_post=_N_POST)
    shp = (rows, _LANES)
    rois_pad = pl.pallas_call(
        body,
        out_shape=jax.ShapeDtypeStruct((_N_POST, _LANES), jnp.float32),
        scratch_shapes=[
            pltpu.VMEM(shp, jnp.float32),
            pltpu.VMEM(shp, jnp.float32),
            pltpu.VMEM(shp, jnp.float32),
            pltpu.VMEM(shp, jnp.float32),
            pltpu.VMEM(shp, jnp.float32),
            pltpu.VMEM(shp, jnp.float32),
            pltpu.VMEM(shp, jnp.int32),
        ],
    )(img_row, lt[0], lt[1], lt[2], lt[3], fg2, at[0], at[1], at[2], at[3])
    return rois_pad[:, :4]


def kernel(x, conv1_w, conv1_b, score_w, score_b, loc_w, loc_b, img_size):
    n, _, hh, ww = x.shape
    anchor = jnp.asarray(_anchors_np(hh, ww))
    features = jax.nn.relu(_conv2d(x, conv1_w, conv1_b, ((1, 1), (1, 1))))
    rpn_locs = _conv2d(features, loc_w, loc_b, 'VALID')
    rpn_locs = jnp.transpose(rpn_locs, (0, 2, 3, 1)).reshape(n, -1, 4)
    rpn_scores = jnp.transpose(_conv2d(features, score_w, score_b, 'VALID'),
                               (0, 2, 3, 1))
    n_anchor = anchor.shape[0] // (hh * ww)
    soft = jax.nn.softmax(rpn_scores.reshape(n, hh, ww, n_anchor, 2), axis=4)
    fg = soft[..., 1].reshape(n, -1)
    rpn_scores = rpn_scores.reshape(n, -1, 2)
    rois = []
    roi_indices = []
    for i in range(n):
        roi = _proposal_pallas(rpn_locs[i], fg[i], anchor, img_size)
        rois.append(roi)
        roi_indices.append(i * jnp.ones((roi.shape[0],), dtype=jnp.int32))
    rois = jnp.stack(rois, axis=0)
    roi_indices = jnp.stack(roi_indices, axis=0)
    return (rpn_locs, rpn_scores, rois, roi_indices, anchor)


# speedup vs baseline: 16.4573x; 1.0269x over previous
import functools

import numpy as np
import jax
import jax.numpy as jnp
from jax.experimental import pallas as pl
from jax.experimental.pallas import tpu as pltpu

_FEAT_STRIDE = 16
_N_POST = 2000
_N_PRE = 12000
_NMS_THRESH = 0.7
_MIN_SIZE = 16.0
_LANES = 128


def _anchor_base_np(base_size=16, ratios=(0.5, 1.0, 2.0), anchor_scales=(8, 16, 32)):
    py = base_size / 2.0
    px = base_size / 2.0
    ab = np.zeros((len(ratios) * len(anchor_scales), 4), dtype=np.float32)
    for i, r in enumerate(ratios):
        for j, s in enumerate(anchor_scales):
            h = base_size * s * np.sqrt(r)
            w = base_size * s * np.sqrt(1.0 / r)
            k = i * len(anchor_scales) + j
            ab[k, 0] = py - h / 2.0
            ab[k, 1] = px - w / 2.0
            ab[k, 2] = py + h / 2.0
            ab[k, 3] = px + w / 2.0
    return ab


def _anchors_np(height, width):
    ab = _anchor_base_np()
    shift_y = np.arange(0, height * _FEAT_STRIDE, _FEAT_STRIDE)
    shift_x = np.arange(0, width * _FEAT_STRIDE, _FEAT_STRIDE)
    sx, sy = np.meshgrid(shift_x, shift_y, indexing='ij')
    shift = np.stack((sy.ravel(), sx.ravel(), sy.ravel(), sx.ravel()), axis=1)
    A = ab.shape[0]
    K = shift.shape[0]
    anchor = ab.reshape((1, A, 4)) + shift.reshape((K, 1, 4))
    return anchor.reshape((K * A, 4)).astype(np.float32)


def _conv2d(x, w, b, pad):
    y = jax.lax.conv_general_dilated(x, w, (1, 1), pad,
                                     dimension_numbers=('NCHW', 'OIHW', 'NCHW'))
    return y + b[None, :, None, None]


def _proposal_body(img_ref, dy_ref, dx_ref, dh_ref, dw_ref, fg_ref,
                   ay1_ref, ax1_ref, ay2_ref, ax2_ref,
                   rois_ref,
                   y1_s, x1_s, y2_s, x2_s, ar_s,
                   *, n_total, n_pre, n_post):
    H = img_ref[0, 0]
    W = img_ref[0, 1]
    rows = y1_s.shape[0]
    blocks = rows // 8
    neg_inf = jnp.float32(-jnp.inf)

    ay1 = ay1_ref[...]
    ax1 = ax1_ref[...]
    src_h = ay2_ref[...] - ay1
    src_w = ax2_ref[...] - ax1
    src_cy = ay1 + 0.5 * src_h
    src_cx = ax1 + 0.5 * src_w
    cy = dy_ref[...] * src_h + src_cy
    cx = dx_ref[...] * src_w + src_cx
    h = jnp.exp(dh_ref[...]) * src_h
    w = jnp.exp(dw_ref[...]) * src_w
    y1 = jnp.clip(cy - 0.5 * h, 0.0, H)
    x1 = jnp.clip(cx - 0.5 * w, 0.0, W)
    y2 = jnp.clip(cy + 0.5 * h, 0.0, H)
    x2 = jnp.clip(cx + 0.5 * w, 0.0, W)
    hs = y2 - y1
    ws = x2 - x1
    ar = hs * ws
    idx = (jax.lax.broadcasted_iota(jnp.int32, (rows, _LANES), 0) * _LANES
           + jax.lax.broadcasted_iota(jnp.int32, (rows, _LANES), 1))
    valid = (hs >= _MIN_SIZE) & (ws >= _MIN_SIZE) & (idx < n_total)
    s = jnp.where(valid, fg_ref[...], neg_inf)

    y1_s[...] = y1
    x1_s[...] = x1
    y2_s[...] = y2
    x2_s[...] = x2
    ar_s[...] = ar

    sbits = jax.lax.bitcast_convert_type(s, jnp.int32)
    key = jnp.where(s >= 0.0, sbits + 1, 0)
    k_target = jnp.int32(n_pre)

    def _bs_val(_, lohi):
        lo, hi = lohi
        mid = (lo + hi) // 2
        cnt = jnp.sum((key >= mid).astype(jnp.int32))
        big = cnt >= k_target
        return jnp.where(big, mid, lo), jnp.where(big, hi, mid)

    lo, _ = jax.lax.fori_loop(0, 31, _bs_val, (jnp.int32(0), jnp.int32((1 << 30) + 2)))
    thr = lo
    cnt_gt = jnp.sum((key > thr).astype(jnp.int32))
    need_eq = k_target - cnt_gt
    ties = key == thr

    def _bs_idx(_, lohi):
        lo2, hi2 = lohi
        mid = (lo2 + hi2) // 2
        cnt = jnp.sum((ties & (idx < mid)).astype(jnp.int32))
        enough = cnt >= need_eq
        return jnp.where(enough, lo2, mid), jnp.where(enough, mid, hi2)

    _, cut = jax.lax.fori_loop(0, 16, _bs_idx,
                               (jnp.int32(0), jnp.int32(rows * _LANES)))
    eligible = (key > thr) | (ties & (idx < cut))
    s0 = jnp.where(eligible, s, neg_inf)

    sub_iota = jax.lax.broadcasted_iota(jnp.int32, (8, _LANES), 0)
    lane8 = jax.lax.broadcasted_iota(jnp.int32, (8, _LANES), 1)
    lane_iota = jax.lax.broadcasted_iota(jnp.int32, (1, _LANES), 1)

    def _fold(sarr):
        s3 = sarr.reshape(blocks, 8, _LANES)
        macc = jnp.full((8, _LANES), neg_inf, jnp.float32)
        racc = jnp.zeros((8, _LANES), jnp.int32)
        for j in range(blocks):
            v = s3[j]
            upd = v > macc
            macc = jnp.where(upd, v, macc)
            racc = jnp.where(upd, j, racc)
        return macc, racc

    macc0, racc0 = _fold(s0)

    def _pick_coord(ref, r, lane):
        row = ref[pl.ds(r, 1), :]
        return jnp.max(jnp.where(lane_iota == lane, row, neg_inf))

    def _nms_iter(i, carry):
        s_cur, macc, racc, fy1, fx1, fy2, fx2 = carry
        m = jnp.max(macc)
        idxv = racc * (8 * _LANES) + sub_iota * _LANES + lane8
        pick = jnp.min(jnp.where(macc == m, idxv, jnp.int32(1 << 30)))
        r = pick // _LANES
        lane = pick - r * _LANES
        by1 = _pick_coord(y1_s, r, lane)
        bx1 = _pick_coord(x1_s, r, lane)
        by2 = _pick_coord(y2_s, r, lane)
        bx2 = _pick_coord(x2_s, r, lane)
        barea = _pick_coord(ar_s, r, lane)

        first = i == 0
        fy1 = jnp.where(first, by1, fy1)
        fx1 = jnp.where(first, bx1, fx1)
        fy2 = jnp.where(first, by2, fy2)
        fx2 = jnp.where(first, bx2, fx2)
        exhausted = m == neg_inf
        oy1 = jnp.where(exhausted, fy1, by1)
        ox1 = jnp.where(exhausted, fx1, bx1)
        oy2 = jnp.where(exhausted, fy2, by2)
        ox2 = jnp.where(exhausted, fx2, bx2)

        row_out = (oy1 * (lane_iota == 0).astype(jnp.float32)
                   + ox1 * (lane_iota == 1).astype(jnp.float32)
                   + oy2 * (lane_iota == 2).astype(jnp.float32)
                   + ox2 * (lane_iota == 3).astype(jnp.float32))
        rois_ref[pl.ds(i, 1), :] = row_out

        yy1 = jnp.maximum(y1, by1)
        xx1 = jnp.maximum(x1, bx1)
        yy2 = jnp.minimum(y2, by2)
        xx2 = jnp.minimum(x2, bx2)
        inter = jnp.maximum(yy2 - yy1, 0.0) * jnp.maximum(xx2 - xx1, 0.0)
        denom = ar + barea - inter + jnp.float32(1e-9)
        kill = inter > jnp.float32(_NMS_THRESH) * denom
        s_new = jnp.where(kill, neg_inf, s_cur)
        macc2, racc2 = _fold(s_new)
        return s_new, macc2, racc2, fy1, fx1, fy2, fx2

    z = jnp.float32(0.0)
    jax.lax.fori_loop(0, n_post, _nms_iter,
                      (s0, macc0, racc0, z, z, z, z))


def _proposal_pallas(locs, fg, anchor, img_size):
    n_total = locs.shape[0]
    rows = -(-n_total // _LANES)
    rows_p = -(-rows // 8) * 8
    pad = rows_p * _LANES - n_total
    n_pre = min(_N_PRE, n_total)

    lt = jnp.pad(locs.T, ((0, 0), (0, pad))).reshape(4, rows_p, _LANES)
    at = jnp.pad(anchor.T, ((0, 0), (0, pad))).reshape(4, rows_p, _LANES)
    fg2 = jnp.pad(fg, (0, pad)).reshape(rows_p, _LANES)
    img_row = jnp.zeros((1, _LANES), jnp.float32).at[0, :2].set(
        img_size.astype(jnp.float32))

    body = functools.partial(_proposal_body, n_total=n_total, n_pre=n_pre,
                             n_post=_N_POST)
    shp = (rows_p, _LANES)
    rois_pad = pl.pallas_call(
        body,
        out_shape=jax.ShapeDtypeStruct((_N_POST, _LANES), jnp.float32),
        scratch_shapes=[
            pltpu.VMEM(shp, jnp.float32),
            pltpu.VMEM(shp, jnp.float32),
            pltpu.VMEM(shp, jnp.float32),
            pltpu.VMEM(shp, jnp.float32),
            pltpu.VMEM(shp, jnp.float32),
        ],
    )(img_row, lt[0], lt[1], lt[2], lt[3], fg2, at[0], at[1], at[2], at[3])
    return rois_pad[:, :4]


def kernel(x, conv1_w, conv1_b, score_w, score_b, loc_w, loc_b, img_size):
    n, _, hh, ww = x.shape
    anchor = jnp.asarray(_anchors_np(hh, ww))
    features = jax.nn.relu(_conv2d(x, conv1_w, conv1_b, ((1, 1), (1, 1))))
    rpn_locs = _conv2d(features, loc_w, loc_b, 'VALID')
    rpn_locs = jnp.transpose(rpn_locs, (0, 2, 3, 1)).reshape(n, -1, 4)
    rpn_scores = jnp.transpose(_conv2d(features, score_w, score_b, 'VALID'),
                               (0, 2, 3, 1))
    n_anchor = anchor.shape[0] // (hh * ww)
    soft = jax.nn.softmax(rpn_scores.reshape(n, hh, ww, n_anchor, 2), axis=4)
    fg = soft[..., 1].reshape(n, -1)
    rpn_scores = rpn_scores.reshape(n, -1, 2)
    rois = []
    roi_indices = []
    for i in range(n):
        roi = _proposal_pallas(rpn_locs[i], fg[i], anchor, img_size)
        rois.append(roi)
        roi_indices.append(i * jnp.ones((roi.shape[0],), dtype=jnp.int32))
    rois = jnp.stack(rois, axis=0)
    roi_indices = jnp.stack(roi_indices, axis=0)
    return (rpn_locs, rpn_scores, rois, roi_indices, anchor)


# speedup vs baseline: 22.7648x; 1.3833x over previous
import functools

import numpy as np
import jax
import jax.numpy as jnp
from jax.experimental import pallas as pl
from jax.experimental.pallas import tpu as pltpu

_FEAT_STRIDE = 16
_N_POST = 2000
_N_PRE = 12000
_NMS_THRESH = 0.7
_MIN_SIZE = 16.0
_LANES = 128


def _anchor_base_np(base_size=16, ratios=(0.5, 1.0, 2.0), anchor_scales=(8, 16, 32)):
    py = base_size / 2.0
    px = base_size / 2.0
    ab = np.zeros((len(ratios) * len(anchor_scales), 4), dtype=np.float32)
    for i, r in enumerate(ratios):
        for j, s in enumerate(anchor_scales):
            h = base_size * s * np.sqrt(r)
            w = base_size * s * np.sqrt(1.0 / r)
            k = i * len(anchor_scales) + j
            ab[k, 0] = py - h / 2.0
            ab[k, 1] = px - w / 2.0
            ab[k, 2] = py + h / 2.0
            ab[k, 3] = px + w / 2.0
    return ab


def _anchors_np(height, width):
    ab = _anchor_base_np()
    shift_y = np.arange(0, height * _FEAT_STRIDE, _FEAT_STRIDE)
    shift_x = np.arange(0, width * _FEAT_STRIDE, _FEAT_STRIDE)
    sx, sy = np.meshgrid(shift_x, shift_y, indexing='ij')
    shift = np.stack((sy.ravel(), sx.ravel(), sy.ravel(), sx.ravel()), axis=1)
    A = ab.shape[0]
    K = shift.shape[0]
    anchor = ab.reshape((1, A, 4)) + shift.reshape((K, 1, 4))
    return anchor.reshape((K * A, 4)).astype(np.float32)


def _conv2d(x, w, b, pad):
    y = jax.lax.conv_general_dilated(x, w, (1, 1), pad,
                                     dimension_numbers=('NCHW', 'OIHW', 'NCHW'))
    return y + b[None, :, None, None]


def _proposal_body(img_ref, dy_ref, dx_ref, dh_ref, dw_ref, fg_ref,
                   ay1_ref, ax1_ref, ay2_ref, ax2_ref,
                   rois_ref,
                   *, n_total, n_pre, n_post):
    H = img_ref[0, 0]
    W = img_ref[0, 1]
    rows = fg_ref.shape[0]
    blocks = rows // 8
    neg_inf = jnp.float32(-jnp.inf)

    ay1 = ay1_ref[...]
    ax1 = ax1_ref[...]
    src_h = ay2_ref[...] - ay1
    src_w = ax2_ref[...] - ax1
    src_cy = ay1 + 0.5 * src_h
    src_cx = ax1 + 0.5 * src_w
    cy = dy_ref[...] * src_h + src_cy
    cx = dx_ref[...] * src_w + src_cx
    h = jnp.exp(dh_ref[...]) * src_h
    w = jnp.exp(dw_ref[...]) * src_w
    y1 = jnp.clip(cy - 0.5 * h, 0.0, H)
    x1 = jnp.clip(cx - 0.5 * w, 0.0, W)
    y2 = jnp.clip(cy + 0.5 * h, 0.0, H)
    x2 = jnp.clip(cx + 0.5 * w, 0.0, W)
    hs = y2 - y1
    ws = x2 - x1
    ar = hs * ws
    idx = (jax.lax.broadcasted_iota(jnp.int32, (rows, _LANES), 0) * _LANES
           + jax.lax.broadcasted_iota(jnp.int32, (rows, _LANES), 1))
    valid = (hs >= _MIN_SIZE) & (ws >= _MIN_SIZE) & (idx < n_total)
    s = jnp.where(valid, fg_ref[...], neg_inf)

    sbits = jax.lax.bitcast_convert_type(s, jnp.int32)
    key = jnp.where(s >= 0.0, sbits + 1, 0)
    k_target = jnp.int32(n_pre)

    def _bs_val(_, lohi):
        lo, hi = lohi
        mid = (lo + hi) // 2
        cnt = jnp.sum((key >= mid).astype(jnp.int32))
        big = cnt >= k_target
        return jnp.where(big, mid, lo), jnp.where(big, hi, mid)

    lo, _ = jax.lax.fori_loop(0, 31, _bs_val, (jnp.int32(0), jnp.int32((1 << 30) + 2)))
    thr = lo
    cnt_gt = jnp.sum((key > thr).astype(jnp.int32))
    need_eq = k_target - cnt_gt
    ties = key == thr

    def _bs_idx(_, lohi):
        lo2, hi2 = lohi
        mid = (lo2 + hi2) // 2
        cnt = jnp.sum((ties & (idx < mid)).astype(jnp.int32))
        enough = cnt >= need_eq
        return jnp.where(enough, lo2, mid), jnp.where(enough, mid, hi2)

    _, cut = jax.lax.fori_loop(0, 16, _bs_idx,
                               (jnp.int32(0), jnp.int32(rows * _LANES)))
    eligible = (key > thr) | (ties & (idx < cut))
    s0 = jnp.where(eligible, s, neg_inf)

    sub_iota = jax.lax.broadcasted_iota(jnp.int32, (8, _LANES), 0)
    lane8 = jax.lax.broadcasted_iota(jnp.int32, (8, _LANES), 1)
    lane_iota = jax.lax.broadcasted_iota(jnp.int32, (1, _LANES), 1)
    base_idx = sub_iota * _LANES + lane8

    y13 = y1.reshape(blocks, 8, _LANES)
    x13 = x1.reshape(blocks, 8, _LANES)
    y23 = y2.reshape(blocks, 8, _LANES)
    x23 = x2.reshape(blocks, 8, _LANES)
    ar3 = ar.reshape(blocks, 8, _LANES)

    def _fold6(sarr):
        s3 = sarr.reshape(blocks, 8, _LANES)
        items = [(s3[j], base_idx + jnp.int32(j * 8 * _LANES),
                  y13[j], x13[j], y23[j], x23[j]) for j in range(blocks)]
        while len(items) > 1:
            nxt = []
            for a, b in zip(items[0::2], items[1::2]):
                better = a[0] >= b[0]
                nxt.append(tuple(jnp.where(better, xa, xb)
                                 for xa, xb in zip(a, b)))
            if len(items) % 2:
                nxt.append(items[-1])
            items = nxt
        return items[0]

    fold0 = _fold6(s0)

    def _combine(a, b):
        better = (a[0] > b[0]) | ((a[0] == b[0]) & (a[1] < b[1]))
        return tuple(jnp.where(better, xa, xb) for xa, xb in zip(a, b))

    def _allreduce6(t):
        for axis, shifts in ((1, [1, 2, 3, 4, 5, 6, 7, 8, 9, 10, 11, 12, 13, 14, 15]),
                             (1, [16, 32, 48, 64, 80, 96, 112]),
                             (0, [1, 2, 3, 4, 5, 6, 7])):
            cands = [t] + [tuple(pltpu.roll(x, k, axis) for x in t)
                           for k in shifts]
            while len(cands) > 1:
                nxt = [_combine(a, b) for a, b in zip(cands[0::2], cands[1::2])]
                if len(cands) % 2:
                    nxt.append(cands[-1])
                cands = nxt
            t = cands[0]
        return t

    def _nms_iter(i, carry):
        s_cur, wv, widx, wy1, wx1, wy2, wx2, fy1, fx1, fy2, fx2 = carry
        bv, _, by1, bx1, by2, bx2 = _allreduce6((wv, widx, wy1, wx1, wy2, wx2))
        barea = (by2 - by1) * (bx2 - bx1)

        first = i == 0
        fy1 = jnp.where(first, by1, fy1)
        fx1 = jnp.where(first, bx1, fx1)
        fy2 = jnp.where(first, by2, fy2)
        fx2 = jnp.where(first, bx2, fx2)
        exhausted = bv == neg_inf
        oy1 = jnp.where(exhausted, fy1, by1)
        ox1 = jnp.where(exhausted, fx1, bx1)
        oy2 = jnp.where(exhausted, fy2, by2)
        ox2 = jnp.where(exhausted, fx2, bx2)

        row_out = (jnp.where(lane_iota == 0, oy1[0:1, :], 0.0)
                   + jnp.where(lane_iota == 1, ox1[0:1, :], 0.0)
                   + jnp.where(lane_iota == 2, oy2[0:1, :], 0.0)
                   + jnp.where(lane_iota == 3, ox2[0:1, :], 0.0))
        rois_ref[pl.ds(i, 1), :] = row_out

        s3c = s_cur.reshape(blocks, 8, _LANES)
        yy1 = jnp.maximum(y13, by1[None])
        xx1 = jnp.maximum(x13, bx1[None])
        yy2 = jnp.minimum(y23, by2[None])
        xx2 = jnp.minimum(x23, bx2[None])
        inter = jnp.maximum(yy2 - yy1, 0.0) * jnp.maximum(xx2 - xx1, 0.0)
        denom = ar3 + barea[None] - inter + jnp.float32(1e-9)
        kill = inter > jnp.float32(_NMS_THRESH) * denom
        s_new = jnp.where(kill, neg_inf, s3c).reshape(rows, _LANES)
        return (s_new,) + _fold6(s_new) + (fy1, fx1, fy2, fx2)

    z = jnp.zeros((8, _LANES), jnp.float32)
    jax.lax.fori_loop(0, n_post, _nms_iter,
                      (s0,) + fold0 + (z, z, z, z))


def _proposal_pallas(locs, fg, anchor, img_size):
    n_total = locs.shape[0]
    rows = -(-n_total // _LANES)
    rows_p = -(-rows // 8) * 8
    pad = rows_p * _LANES - n_total
    n_pre = min(_N_PRE, n_total)

    lt = jnp.pad(locs.T, ((0, 0), (0, pad))).reshape(4, rows_p, _LANES)
    at = jnp.pad(anchor.T, ((0, 0), (0, pad))).reshape(4, rows_p, _LANES)
    fg2 = jnp.pad(fg, (0, pad)).reshape(rows_p, _LANES)
    img_row = jnp.zeros((1, _LANES), jnp.float32).at[0, :2].set(
        img_size.astype(jnp.float32))

    body = functools.partial(_proposal_body, n_total=n_total, n_pre=n_pre,
                             n_post=_N_POST)
    rois_pad = pl.pallas_call(
        body,
        out_shape=jax.ShapeDtypeStruct((_N_POST, _LANES), jnp.float32),
    )(img_row, lt[0], lt[1], lt[2], lt[3], fg2, at[0], at[1], at[2], at[3])
    return rois_pad[:, :4]


def kernel(x, conv1_w, conv1_b, score_w, score_b, loc_w, loc_b, img_size):
    n, _, hh, ww = x.shape
    anchor = jnp.asarray(_anchors_np(hh, ww))
    features = jax.nn.relu(_conv2d(x, conv1_w, conv1_b, ((1, 1), (1, 1))))
    rpn_locs = _conv2d(features, loc_w, loc_b, 'VALID')
    rpn_locs = jnp.transpose(rpn_locs, (0, 2, 3, 1)).reshape(n, -1, 4)
    rpn_scores = jnp.transpose(_conv2d(features, score_w, score_b, 'VALID'),
                               (0, 2, 3, 1))
    n_anchor = anchor.shape[0] // (hh * ww)
    soft = jax.nn.softmax(rpn_scores.reshape(n, hh, ww, n_anchor, 2), axis=4)
    fg = soft[..., 1].reshape(n, -1)
    rpn_scores = rpn_scores.reshape(n, -1, 2)
    rois = []
    roi_indices = []
    for i in range(n):
        roi = _proposal_pallas(rpn_locs[i], fg[i], anchor, img_size)
        rois.append(roi)
        roi_indices.append(i * jnp.ones((roi.shape[0],), dtype=jnp.int32))
    rois = jnp.stack(rois, axis=0)
    roi_indices = jnp.stack(roi_indices, axis=0)
    return (rpn_locs, rpn_scores, rois, roi_indices, anchor)
